# baseline (speedup 1.0000x reference)
import functools

import jax
import jax.numpy as jnp
from jax import lax
from jax.experimental import pallas as pl
from jax.experimental.pallas import tpu as pltpu
from jax.experimental.pallas import tpu_sc as plsc

NC = 2
NS = 16
NW = NC * NS

L = 128
TOTAL_N = 327680
ROWS = TOTAL_N // L
ROWS_PER_W = ROWS // NW

R = 2
D0 = 64
D1 = 32
TILE = 8 * L


def _pipe_feature(tab, vals, out, idx_v, rows_v, tbuf, gsems, wsems, r, d,
                  base):
  nblk = ROWS_PER_W // r
  npair = nblk // 2
  fbs = d // 8
  iota = lax.iota(jnp.int32, 16)

  def stage(b, s):
    off = base + b * r
    pltpu.sync_copy(vals.at[pl.ds(off, r)], idx_v.at[s])
    for j in range(r):
      pltpu.async_copy(
          tab.at[idx_v.at[s, j]], rows_v.at[s, pl.ds(j * L, L)], gsems[s])

  def waitg(s):
    pltpu.make_async_copy(
        tab.at[pl.ds(0, r * L)], rows_v.at[s], gsems[s]).wait()

  def waitw(s):
    pltpu.make_async_copy(
        out.at[:, pl.ds(0, r)], tbuf.at[s], wsems[s]).wait()

  def transpose(s):
    slot = jnp.full((16,), s, jnp.int32)

    def step(c8, carry):
      for k in range(r):
        idx0 = 128 * k + 16 * c8 + iota
        for fb in range(fbs):
          for fi in range(8):
            col = jnp.full((16,), 8 * fb + fi, jnp.int32)
            v = plsc.load_gather(rows_v, [slot, idx0, col])
            tbuf.at[s, fb, k, fi][pl.ds(16 * c8, 16)] = v
      return carry

    lax.fori_loop(0, L // 16, step, 0, unroll=False)

  def wb(b, s):
    off = base + b * r
    for fb in range(fbs):
      pltpu.async_copy(
          tbuf.at[s, fb], out.at[fb, pl.ds(off, r)], wsems[s])

  stage(0, 0)

  def pair(k, carry):
    b0 = 2 * k

    stage(b0 + 1, 1)
    waitg(0)

    @pl.when(k > 0)
    def _():
      waitw(0)

    transpose(0)
    wb(b0, 0)

    @pl.when(k < npair - 1)
    def _():
      stage(b0 + 2, 0)

    waitg(1)

    @pl.when(k > 0)
    def _():
      waitw(1)

    transpose(1)
    wb(b0 + 1, 1)
    return carry

  lax.fori_loop(0, npair, pair, 0, unroll=False)
  waitw(0)
  waitw(1)


def _emb_body(t0, t1, v0, v1, o0, o1, idx0, rows0, tbuf0, idx1, rows1, tbuf1,
              gsem0, gsem1, wsem0, wsem1):
  wid = lax.axis_index("s") * NC + lax.axis_index("c")
  base = wid * ROWS_PER_W
  gsems = (gsem0, gsem1)
  wsems = (wsem0, wsem1)
  _pipe_feature(t0, v0, o0, idx0, rows0, tbuf0, gsems, wsems, R, D0, base)
  _pipe_feature(t1, v1, o1, idx1, rows1, tbuf1, gsems, wsems, R, D1, base)


@functools.cache
def _build():
  mesh = plsc.VectorSubcoreMesh(
      core_axis_name="c", subcore_axis_name="s",
      num_cores=NC, num_subcores=NS)
  return pl.kernel(
      _emb_body,
      out_type=(
          jax.ShapeDtypeStruct((D0 // 8, ROWS, 8, L), jnp.float32),
          jax.ShapeDtypeStruct((D1 // 8, ROWS, 8, L), jnp.float32),
      ),
      mesh=mesh,
      scratch_types=[
          pltpu.VMEM((2, R, L), jnp.int32),
          pltpu.VMEM((2, R * L, D0), jnp.float32),
          pltpu.VMEM((2, D0 // 8, R, 8, L), jnp.float32),
          pltpu.VMEM((2, R, L), jnp.int32),
          pltpu.VMEM((2, R * L, D1), jnp.float32),
          pltpu.VMEM((2, D1 // 8, R, 8, L), jnp.float32),
          pltpu.SemaphoreType.DMA,
          pltpu.SemaphoreType.DMA,
          pltpu.SemaphoreType.DMA,
          pltpu.SemaphoreType.DMA,
      ],
      compiler_params=pltpu.CompilerParams(
          use_tc_tiling_on_sc=False, needs_layout_passes=False),
  )


def kernel(table_t0, table_t1, values_f0, values_f1, lengths_f0, lengths_f1):
  v0 = values_f0.reshape(ROWS, L)
  v1 = values_f1.reshape(ROWS, L)
  out0, out1 = _build()(table_t0, table_t1, v0, v1)
  emb0 = out0.transpose(1, 3, 0, 2).reshape(TOTAL_N, D0)
  emb1 = out1.transpose(1, 3, 0, 2).reshape(TOTAL_N, D1)
  return (emb0, emb1)


# speedup vs baseline: 1.3520x; 1.3520x over previous
import functools

import jax
import jax.numpy as jnp
from jax import lax
from jax.experimental import pallas as pl
from jax.experimental.pallas import tpu as pltpu
from jax.experimental.pallas import tpu_sc as plsc

NC = 2
NS = 16
NW = NC * NS

L = 128
TOTAL_N = 327680
ROWS = TOTAL_N // L
ROWS_PER_W = ROWS // NW

R0 = 4
R1 = 5

D0 = 64
D1 = 32


def _pipe_feature(tab, vals, out, idx_v, rows_v, gsems, wsems, r, d, base):
  nblk = ROWS_PER_W // r
  npair = nblk // 2

  def stage(b, s):
    off = base + b * r
    pltpu.sync_copy(vals.at[pl.ds(off, r)], idx_v.at[s])
    for j in range(r):
      pltpu.async_copy(
          tab.at[idx_v.at[s, j]], rows_v.at[s, pl.ds(j * L, L)], gsems[s])

  def waitg(s):
    pltpu.make_async_copy(
        out.at[pl.ds(base * L, r * L)], rows_v.at[s], gsems[s]).wait()

  def waitw(s):
    pltpu.make_async_copy(
        out.at[pl.ds(base * L, r * L)], rows_v.at[s], wsems[s]).wait()

  def wb(b, s):
    off = base + b * r
    pltpu.async_copy(rows_v.at[s], out.at[pl.ds(off * L, r * L)], wsems[s])

  stage(0, 0)

  def pair(k, carry):
    b0 = 2 * k

    @pl.when(k > 0)
    def _():
      waitw(1)

    stage(b0 + 1, 1)
    waitg(0)
    wb(b0, 0)

    @pl.when(k < npair - 1)
    def _():
      waitw(0)
      stage(b0 + 2, 0)

    waitg(1)
    wb(b0 + 1, 1)
    return carry

  lax.fori_loop(0, npair, pair, 0, unroll=False)
  waitw(0)
  waitw(1)


def _emb_body(t0, t1, v0, v1, o0, o1, idx0, rows0, idx1, rows1,
              gsem0, gsem1, wsem0, wsem1):
  wid = lax.axis_index("s") * NC + lax.axis_index("c")
  base = wid * ROWS_PER_W
  gsems = (gsem0, gsem1)
  wsems = (wsem0, wsem1)
  _pipe_feature(t0, v0, o0, idx0, rows0, gsems, wsems, R0, D0, base)
  _pipe_feature(t1, v1, o1, idx1, rows1, gsems, wsems, R1, D1, base)


@functools.cache
def _build():
  mesh = plsc.VectorSubcoreMesh(
      core_axis_name="c", subcore_axis_name="s",
      num_cores=NC, num_subcores=NS)
  return pl.kernel(
      _emb_body,
      out_type=(
          jax.ShapeDtypeStruct((TOTAL_N, D0), jnp.float32),
          jax.ShapeDtypeStruct((TOTAL_N, D1), jnp.float32),
      ),
      mesh=mesh,
      scratch_types=[
          pltpu.VMEM((2, R0, L), jnp.int32),
          pltpu.VMEM((2, R0 * L, D0), jnp.float32),
          pltpu.VMEM((2, R1, L), jnp.int32),
          pltpu.VMEM((2, R1 * L, D1), jnp.float32),
          pltpu.SemaphoreType.DMA,
          pltpu.SemaphoreType.DMA,
          pltpu.SemaphoreType.DMA,
          pltpu.SemaphoreType.DMA,
      ],
      compiler_params=pltpu.CompilerParams(use_tc_tiling_on_sc=False),
  )


def kernel(table_t0, table_t1, values_f0, values_f1, lengths_f0, lengths_f1):
  v0 = values_f0.reshape(ROWS, L)
  v1 = values_f1.reshape(ROWS, L)
  out0, out1 = _build()(table_t0, table_t1, v0, v1)
  return (out0, out1)
